# baseline (speedup 1.0000x reference)
import functools
import math

import jax
import jax.numpy as jnp
from jax import lax
from jax.experimental import pallas as pl
from jax.experimental.pallas import tpu as pltpu
from jax.experimental.pallas import tpu_sc as plsc

K = 1024
D = 32
LMBDA = 0.05
DA = 5


def _vq_body(x_ref, sx2_ref, e2x_ref, eaug_ref, e2_ref, bias_ref,
             ind_ref, stats_ref):
    n = pl.program_id(0)
    hb = pl.program_id(1)
    W = x_ref.shape[2]
    xb = x_ref[0]
    mm2 = jnp.dot(e2x_ref[...], xb, preferred_element_type=jnp.float32)
    dist = ((sx2_ref[0] + e2_ref[...]) - mm2) + bias_ref[...]
    dmin = jnp.min(dist, axis=0, keepdims=True)
    eq = dist == dmin
    ohm = eq.astype(jnp.float32)
    aug = jnp.dot(eaug_ref[...], ohm, preferred_element_type=jnp.float32)
    ties = jnp.max(aug[DA - 1:DA, :]) > 1.5

    @pl.when((n == 0) & (hb == 0))
    def _():
        stats_ref[...] = jnp.zeros_like(stats_ref)

    def _finish(a, ind):
        ind_ref[...] = ind[None]
        mse_p = jnp.sum(dmin - a[3:4, :])
        rate_p = jnp.sum(a[2:3, :])
        sio = jax.lax.broadcasted_iota(jnp.int32, (8, 128), 0)
        stats_ref[...] += (jnp.where(sio == 0, mse_p, 0.0)
                           + jnp.where(sio == 1, rate_p, 0.0))

    @pl.when(jnp.logical_not(ties))
    def _():
        ind = (aug[0:1, :] * 256.0 + aug[1:2, :]).astype(jnp.int32)
        _finish(aug, ind)

    @pl.when(ties)
    def _():
        kio = jax.lax.broadcasted_iota(jnp.int32, (K, W), 0)
        ind = jnp.min(jnp.where(eq, kio, K), axis=0, keepdims=True)
        oh = (kio == ind).astype(jnp.float32)
        aug2 = jnp.dot(eaug_ref[...], oh, preferred_element_type=jnp.float32)
        _finish(aug2, ind)


def _make_sc_gather(B, b_per_w, NC):
    mesh = plsc.VectorSubcoreMesh(core_axis_name="c", subcore_axis_name="s")

    @functools.partial(
        pl.kernel, mesh=mesh,
        out_type=jax.ShapeDtypeStruct((B, D), jnp.float32),
        compiler_params=pltpu.CompilerParams(use_tc_tiling_on_sc=False),
        scratch_types=[
            pltpu.VMEM((b_per_w,), jnp.int32),
            pltpu.VMEM((b_per_w, D), jnp.float32),
            pltpu.SemaphoreType.DMA,
        ],
    )
    def gather_k(table_hbm, idx_hbm, out_hbm, idx_v, rows_v, sem):
        wid = lax.axis_index("s") * NC + lax.axis_index("c")
        base = wid * b_per_w
        pltpu.sync_copy(idx_hbm.at[pl.ds(base, b_per_w)], idx_v)
        pltpu.async_copy(table_hbm.at[idx_v], rows_v, sem).wait()
        pltpu.sync_copy(rows_v, out_hbm.at[pl.ds(base, b_per_w)])

    return gather_k


def kernel(latents, embedding_weight, pmf_logits):
    N, H, W = latents.shape
    target_rows = H % D
    if target_rows != 0:
        pad_len = D - target_rows
        latents_e = jnp.concatenate([latents, latents[:, -pad_len:, :]], axis=1)
    else:
        latents_e = latents
    Hp = latents_e.shape[1]
    HB = Hp // D
    M = N * W * HB

    sx2_g = jnp.sum(latents_e.reshape(N, HB, D, W) ** 2,
                    axis=2).reshape(N * HB, 1, W)
    e2 = jnp.sum(embedding_weight ** 2, axis=1)[:, None]
    log_pmf = jax.nn.log_softmax(pmf_logits)
    log2_pmf = log_pmf / -math.log(2.0)
    rate_bias_r = log2_pmf / LMBDA
    rate_bias = rate_bias_r[:, None]
    kk = jnp.arange(K, dtype=jnp.int32)
    eaug = jnp.concatenate([
        (kk // 256).astype(jnp.float32)[None, :],
        (kk % 256).astype(jnp.float32)[None, :],
        log2_pmf[None, :],
        rate_bias_r[None, :],
        jnp.ones((1, K), jnp.float32),
    ], axis=0)

    inds_g, stats = pl.pallas_call(
        _vq_body,
        grid=(N, HB),
        in_specs=[
            pl.BlockSpec((1, D, W), lambda n, hb: (n, hb, 0)),
            pl.BlockSpec((1, 1, W), lambda n, hb: (n * HB + hb, 0, 0)),
            pl.BlockSpec((K, D), lambda n, hb: (0, 0)),
            pl.BlockSpec((DA, K), lambda n, hb: (0, 0)),
            pl.BlockSpec((K, 1), lambda n, hb: (0, 0)),
            pl.BlockSpec((K, 1), lambda n, hb: (0, 0)),
        ],
        out_specs=[
            pl.BlockSpec((1, 1, W), lambda n, hb: (n * HB + hb, 0, 0)),
            pl.BlockSpec((8, 128), lambda n, hb: (0, 0)),
        ],
        out_shape=[
            jax.ShapeDtypeStruct((N * HB, 1, W), jnp.int32),
            jax.ShapeDtypeStruct((8, 128), jnp.float32),
        ],
    )(latents_e, sx2_g, embedding_weight * 2.0, eaug, e2, rate_bias)

    idx_flat = inds_g.reshape(M)
    NW = 32
    gather_k = _make_sc_gather(M, M // NW, 2)
    rows = gather_k(embedding_weight, idx_flat)
    quantized = jnp.transpose(rows.reshape(N, HB, W, D),
                              (0, 1, 3, 2)).reshape(N, Hp, W)[:, :H, :]
    inds = jnp.transpose(inds_g.reshape(N, HB, W), (0, 2, 1)).reshape(M, 1)
    mse_loss = stats[0, 0] / jnp.float32(M * D)
    rate_uem = stats[1, 0]
    prior_dist = jnp.zeros(1, dtype=jnp.float32)
    param_bit = jnp.zeros(1, dtype=jnp.float32)
    return (quantized, mse_loss, inds, rate_uem, prior_dist, param_bit)


# speedup vs baseline: 2.5464x; 2.5464x over previous
import functools
import math

import jax
import jax.numpy as jnp
from jax import lax
from jax.experimental import pallas as pl
from jax.experimental.pallas import tpu as pltpu
from jax.experimental.pallas import tpu_sc as plsc

K = 1024
D = 32
LMBDA = 0.05
DA = 5


def _vq_body(x_ref, sx2_ref, e2x_ref, eaug_ref, e2_ref, bias_ref,
             ind_ref, stats_ref):
    n = pl.program_id(0)
    hb = pl.program_id(1)
    W = x_ref.shape[2]
    xb = x_ref[0]
    mm2 = jnp.dot(e2x_ref[...], xb, preferred_element_type=jnp.float32)
    dist = ((sx2_ref[0] + e2_ref[...]) - mm2) + bias_ref[...]
    dmin = jnp.min(dist, axis=0, keepdims=True)
    eq = dist == dmin
    ohm = eq.astype(jnp.float32)
    aug = jnp.dot(eaug_ref[...], ohm, preferred_element_type=jnp.float32)
    ties = jnp.max(aug[DA - 1:DA, :]) > 1.5

    @pl.when((n == 0) & (hb == 0))
    def _():
        stats_ref[...] = jnp.zeros_like(stats_ref)

    def _finish(a, ind):
        ind_ref[...] = ind[None]
        mse_p = jnp.sum(dmin - a[3:4, :])
        rate_p = jnp.sum(a[2:3, :])
        sio = jax.lax.broadcasted_iota(jnp.int32, (8, 128), 0)
        stats_ref[...] += (jnp.where(sio == 0, mse_p, 0.0)
                           + jnp.where(sio == 1, rate_p, 0.0))

    @pl.when(jnp.logical_not(ties))
    def _():
        ind = (aug[0:1, :] * 256.0 + aug[1:2, :]).astype(jnp.int32)
        _finish(aug, ind)

    @pl.when(ties)
    def _():
        kio = jax.lax.broadcasted_iota(jnp.int32, (K, W), 0)
        ind = jnp.min(jnp.where(eq, kio, K), axis=0, keepdims=True)
        oh = (kio == ind).astype(jnp.float32)
        aug2 = jnp.dot(eaug_ref[...], oh, preferred_element_type=jnp.float32)
        _finish(aug2, ind)


def _make_sc_gather(n_slabs, W):
    info = plsc.get_sparse_core_info()
    NC, NS, L = info.num_cores, info.num_subcores, info.num_lanes
    NW = NC * NS
    slabs_per_sub = n_slabs // NW
    tok_per_sub = slabs_per_sub * W
    mesh = plsc.VectorSubcoreMesh(core_axis_name="c", subcore_axis_name="s")

    @functools.partial(
        pl.kernel, mesh=mesh,
        out_type=jax.ShapeDtypeStruct((n_slabs, D, W), jnp.float32),
        compiler_params=pltpu.CompilerParams(use_tc_tiling_on_sc=False,
                                             needs_layout_passes=False),
        scratch_types=[
            pltpu.VMEM((D, K), jnp.float32),
            pltpu.VMEM((tok_per_sub,), jnp.int32),
            pltpu.VMEM((D, W), jnp.float32),
        ],
    )
    def gather_k(et_hbm, idx_hbm, out_hbm, tab_v, idx_v, out_v):
        wid = lax.axis_index("s") * NC + lax.axis_index("c")
        pltpu.sync_copy(et_hbm, tab_v)
        pltpu.sync_copy(idx_hbm.at[pl.ds(wid * tok_per_sub, tok_per_sub)], idx_v)
        for t in range(slabs_per_sub):
            for g in range(W // L):
                idx16 = idx_v[pl.ds(t * W + g * L, L)]
                for d in range(D):
                    dsplat = jnp.full((L,), d, jnp.int32)
                    out_v[d, pl.ds(g * L, L)] = plsc.load_gather(
                        tab_v, [dsplat, idx16])
            pltpu.sync_copy(out_v, out_hbm.at[wid * slabs_per_sub + t])

    return gather_k


def kernel(latents, embedding_weight, pmf_logits):
    N, H, W = latents.shape
    target_rows = H % D
    if target_rows != 0:
        pad_len = D - target_rows
        latents_e = jnp.concatenate([latents, latents[:, -pad_len:, :]], axis=1)
    else:
        latents_e = latents
    Hp = latents_e.shape[1]
    HB = Hp // D
    M = N * W * HB

    sx2_g = jnp.sum(latents_e.reshape(N, HB, D, W) ** 2,
                    axis=2).reshape(N * HB, 1, W)
    e2 = jnp.sum(embedding_weight ** 2, axis=1)[:, None]
    log_pmf = jax.nn.log_softmax(pmf_logits)
    log2_pmf = log_pmf / -math.log(2.0)
    rate_bias_r = log2_pmf / LMBDA
    rate_bias = rate_bias_r[:, None]
    kk = jnp.arange(K, dtype=jnp.int32)
    eaug = jnp.concatenate([
        (kk // 256).astype(jnp.float32)[None, :],
        (kk % 256).astype(jnp.float32)[None, :],
        log2_pmf[None, :],
        rate_bias_r[None, :],
        jnp.ones((1, K), jnp.float32),
    ], axis=0)

    inds_g, stats = pl.pallas_call(
        _vq_body,
        grid=(N, HB),
        in_specs=[
            pl.BlockSpec((1, D, W), lambda n, hb: (n, hb, 0)),
            pl.BlockSpec((1, 1, W), lambda n, hb: (n * HB + hb, 0, 0)),
            pl.BlockSpec((K, D), lambda n, hb: (0, 0)),
            pl.BlockSpec((DA, K), lambda n, hb: (0, 0)),
            pl.BlockSpec((K, 1), lambda n, hb: (0, 0)),
            pl.BlockSpec((K, 1), lambda n, hb: (0, 0)),
        ],
        out_specs=[
            pl.BlockSpec((1, 1, W), lambda n, hb: (n * HB + hb, 0, 0)),
            pl.BlockSpec((8, 128), lambda n, hb: (0, 0)),
        ],
        out_shape=[
            jax.ShapeDtypeStruct((N * HB, 1, W), jnp.int32),
            jax.ShapeDtypeStruct((8, 128), jnp.float32),
        ],
    )(latents_e, sx2_g, embedding_weight * 2.0, eaug, e2, rate_bias)

    gather_k = _make_sc_gather(N * HB, W)
    qs = gather_k(embedding_weight.T, inds_g.reshape(M))
    quantized = qs.reshape(N, Hp, W)[:, :H, :]
    inds = jnp.transpose(inds_g.reshape(N, HB, W), (0, 2, 1)).reshape(M, 1)
    mse_loss = stats[0, 0] / jnp.float32(M * D)
    rate_uem = stats[1, 0]
    prior_dist = jnp.zeros(1, dtype=jnp.float32)
    param_bit = jnp.zeros(1, dtype=jnp.float32)
    return (quantized, mse_loss, inds, rate_uem, prior_dist, param_bit)


# speedup vs baseline: 3.2524x; 1.2773x over previous
import math

import jax
import jax.numpy as jnp
from jax.experimental import pallas as pl
from jax.experimental.pallas import tpu as pltpu

K = 1024
D = 32
LMBDA = 0.05
DA = D + 4
SPG = 2


def _vq_body(x_ref, sx2_ref, e2x_ref, eaug_ref, e2_ref, bias_ref,
             q_ref, ind_ref, stats_ref):
    n = pl.program_id(0)
    g = pl.program_id(1)
    W = x_ref.shape[2]

    @pl.when((n == 0) & (g == 0))
    def _():
        stats_ref[...] = jnp.zeros_like(stats_ref)

    for j in range(SPG):
        xb = x_ref[0, j * D:(j + 1) * D, :]
        mm2 = jnp.dot(e2x_ref[...], xb,
                      preferred_element_type=jnp.float32)
        dist = ((sx2_ref[0, j:j + 1, :] + e2_ref[...]) - mm2) + bias_ref[...]
        dmin = jnp.min(dist, axis=0, keepdims=True)
        eq = dist == dmin
        ohm = eq.astype(jnp.float32)
        aug = jnp.dot(eaug_ref[...], ohm, preferred_element_type=jnp.float32)
        ties = jnp.max(aug[DA - 1:DA, :]) > 1.5

        def _finish(a, ind, xb=xb, j=j):
            qt = a[:D]
            q_ref[0, j * D:(j + 1) * D, :] = qt
            ind_ref[0, j:j + 1, :] = ind
            mse_p = jnp.sum((qt - xb) ** 2)
            rate_p = jnp.sum(a[D + 2:D + 3, :])
            sio = jax.lax.broadcasted_iota(jnp.int32, (8, 128), 0)
            stats_ref[...] += (jnp.where(sio == 0, mse_p, 0.0)
                               + jnp.where(sio == 1, rate_p, 0.0))

        @pl.when(jnp.logical_not(ties))
        def _(aug=aug, _finish=_finish):
            ind = (aug[D:D + 1, :] * 256.0
                   + aug[D + 1:D + 2, :]).astype(jnp.int32)
            _finish(aug, ind)

        @pl.when(ties)
        def _(eq=eq, _finish=_finish):
            kio = jax.lax.broadcasted_iota(jnp.int32, (K, W), 0)
            ind = jnp.min(jnp.where(eq, kio, K), axis=0, keepdims=True)
            oh = (kio == ind).astype(jnp.float32)
            aug2 = jnp.dot(eaug_ref[...], oh,
                           preferred_element_type=jnp.float32)
            _finish(aug2, ind)


def kernel(latents, embedding_weight, pmf_logits):
    N, H, W = latents.shape
    target_rows = H % D
    if target_rows != 0:
        pad_len = D - target_rows
        latents_e = jnp.concatenate([latents, latents[:, -pad_len:, :]], axis=1)
    else:
        latents_e = latents
    Hp = latents_e.shape[1]
    HB = Hp // D
    M = N * W * HB
    G = HB // SPG

    sx2_g = jnp.sum(latents_e.reshape(N, HB, D, W) ** 2,
                    axis=2).reshape(N * G, SPG, W)
    e2 = jnp.sum(embedding_weight ** 2, axis=1)[:, None]
    log_pmf = jax.nn.log_softmax(pmf_logits)
    log2_pmf = log_pmf / -math.log(2.0)
    rate_bias = (log2_pmf / LMBDA)[:, None]
    kk = jnp.arange(K, dtype=jnp.int32)
    eaug = jnp.concatenate([
        embedding_weight.T,
        (kk // 256).astype(jnp.float32)[None, :],
        (kk % 256).astype(jnp.float32)[None, :],
        log2_pmf[None, :],
        jnp.ones((1, K), jnp.float32),
    ], axis=0)

    qe, inds_g, stats = pl.pallas_call(
        _vq_body,
        grid=(N, G),
        in_specs=[
            pl.BlockSpec((1, SPG * D, W), lambda n, g: (n, g, 0)),
            pl.BlockSpec((1, SPG, W), lambda n, g: (n * G + g, 0, 0)),
            pl.BlockSpec((K, D), lambda n, g: (0, 0)),
            pl.BlockSpec((DA, K), lambda n, g: (0, 0)),
            pl.BlockSpec((K, 1), lambda n, g: (0, 0)),
            pl.BlockSpec((K, 1), lambda n, g: (0, 0)),
        ],
        out_specs=[
            pl.BlockSpec((1, SPG * D, W), lambda n, g: (n, g, 0)),
            pl.BlockSpec((1, SPG, W), lambda n, g: (n * G + g, 0, 0)),
            pl.BlockSpec((8, 128), lambda n, g: (0, 0)),
        ],
        out_shape=[
            jax.ShapeDtypeStruct((N, Hp, W), jnp.float32),
            jax.ShapeDtypeStruct((N * G, SPG, W), jnp.int32),
            jax.ShapeDtypeStruct((8, 128), jnp.float32),
        ],
    )(latents_e, sx2_g, embedding_weight * 2.0, eaug, e2, rate_bias)

    quantized = qe[:, :H, :]
    inds = jnp.transpose(inds_g.reshape(N, HB, W), (0, 2, 1)).reshape(M, 1)
    mse_loss = stats[0, 0] / jnp.float32(M * D)
    rate_uem = stats[1, 0]
    prior_dist = jnp.zeros(1, dtype=jnp.float32)
    param_bit = jnp.zeros(1, dtype=jnp.float32)
    return (quantized, mse_loss, inds, rate_uem, prior_dist, param_bit)


# speedup vs baseline: 3.3292x; 1.0236x over previous
import math

import jax
import jax.numpy as jnp
from jax.experimental import pallas as pl
from jax.experimental.pallas import tpu as pltpu

K = 1024
D = 32
LMBDA = 0.05
DA = D + 4
SPG = 4


def _vq_body(x_ref, sx2_ref, e2x_ref, eaug_ref, e2_ref, bias_ref,
             q_ref, ind_ref, stats_ref):
    n = pl.program_id(0)
    g = pl.program_id(1)
    W = x_ref.shape[2]

    @pl.when((n == 0) & (g == 0))
    def _():
        stats_ref[...] = jnp.zeros_like(stats_ref)

    for j in range(SPG):
        xb = x_ref[0, j * D:(j + 1) * D, :]
        mm2 = jnp.dot(e2x_ref[...], xb,
                      preferred_element_type=jnp.float32)
        dist = ((sx2_ref[0, j:j + 1, :] + e2_ref[...]) - mm2) + bias_ref[...]
        dmin = jnp.min(dist, axis=0, keepdims=True)
        eq = dist == dmin
        ohm = eq.astype(jnp.float32)
        aug = jnp.dot(eaug_ref[...], ohm, preferred_element_type=jnp.float32)
        ties = jnp.max(aug[DA - 1:DA, :]) > 1.5

        def _finish(a, ind, xb=xb, j=j):
            qt = a[:D]
            q_ref[0, j * D:(j + 1) * D, :] = qt
            ind_ref[0, j:j + 1, :] = ind
            mse_p = jnp.sum((qt - xb) ** 2)
            rate_p = jnp.sum(a[D + 2:D + 3, :])
            sio = jax.lax.broadcasted_iota(jnp.int32, (8, 128), 0)
            stats_ref[...] += (jnp.where(sio == 0, mse_p, 0.0)
                               + jnp.where(sio == 1, rate_p, 0.0))

        @pl.when(jnp.logical_not(ties))
        def _(aug=aug, _finish=_finish):
            ind = (aug[D:D + 1, :] * 256.0
                   + aug[D + 1:D + 2, :]).astype(jnp.int32)
            _finish(aug, ind)

        @pl.when(ties)
        def _(eq=eq, _finish=_finish):
            kio = jax.lax.broadcasted_iota(jnp.int32, (K, W), 0)
            ind = jnp.min(jnp.where(eq, kio, K), axis=0, keepdims=True)
            oh = (kio == ind).astype(jnp.float32)
            aug2 = jnp.dot(eaug_ref[...], oh,
                           preferred_element_type=jnp.float32)
            _finish(aug2, ind)


def kernel(latents, embedding_weight, pmf_logits):
    N, H, W = latents.shape
    target_rows = H % D
    if target_rows != 0:
        pad_len = D - target_rows
        latents_e = jnp.concatenate([latents, latents[:, -pad_len:, :]], axis=1)
    else:
        latents_e = latents
    Hp = latents_e.shape[1]
    HB = Hp // D
    M = N * W * HB
    G = HB // SPG

    sx2_g = jnp.sum(latents_e.reshape(N, HB, D, W) ** 2,
                    axis=2).reshape(N * G, SPG, W)
    e2 = jnp.sum(embedding_weight ** 2, axis=1)[:, None]
    log_pmf = jax.nn.log_softmax(pmf_logits)
    log2_pmf = log_pmf / -math.log(2.0)
    rate_bias = (log2_pmf / LMBDA)[:, None]
    kk = jnp.arange(K, dtype=jnp.int32)
    eaug = jnp.concatenate([
        embedding_weight.T,
        (kk // 256).astype(jnp.float32)[None, :],
        (kk % 256).astype(jnp.float32)[None, :],
        log2_pmf[None, :],
        jnp.ones((1, K), jnp.float32),
    ], axis=0)

    qe, inds_g, stats = pl.pallas_call(
        _vq_body,
        grid=(N, G),
        in_specs=[
            pl.BlockSpec((1, SPG * D, W), lambda n, g: (n, g, 0)),
            pl.BlockSpec((1, SPG, W), lambda n, g: (n * G + g, 0, 0)),
            pl.BlockSpec((K, D), lambda n, g: (0, 0)),
            pl.BlockSpec((DA, K), lambda n, g: (0, 0)),
            pl.BlockSpec((K, 1), lambda n, g: (0, 0)),
            pl.BlockSpec((K, 1), lambda n, g: (0, 0)),
        ],
        out_specs=[
            pl.BlockSpec((1, SPG * D, W), lambda n, g: (n, g, 0)),
            pl.BlockSpec((1, SPG, W), lambda n, g: (n * G + g, 0, 0)),
            pl.BlockSpec((8, 128), lambda n, g: (0, 0)),
        ],
        out_shape=[
            jax.ShapeDtypeStruct((N, Hp, W), jnp.float32),
            jax.ShapeDtypeStruct((N * G, SPG, W), jnp.int32),
            jax.ShapeDtypeStruct((8, 128), jnp.float32),
        ],
    )(latents_e, sx2_g, embedding_weight * 2.0, eaug, e2, rate_bias)

    quantized = qe[:, :H, :]
    inds = jnp.transpose(inds_g.reshape(N, HB, W), (0, 2, 1)).reshape(M, 1)
    mse_loss = stats[0, 0] / jnp.float32(M * D)
    rate_uem = stats[1, 0]
    prior_dist = jnp.zeros(1, dtype=jnp.float32)
    param_bit = jnp.zeros(1, dtype=jnp.float32)
    return (quantized, mse_loss, inds, rate_uem, prior_dist, param_bit)
